# baseline (speedup 1.0000x reference)
import jax
import jax.numpy as jnp
from jax.experimental import pallas as pl
from jax.experimental.pallas import tpu as pltpu

_BB = 4
_NSLOT = 8


def _onehot_dma(idx_ref, out_ref, buf_ref, *sems):
    j = pl.program_id(0)
    nsteps = pl.num_programs(0)
    slot = jax.lax.rem(j, _NSLOT)

    @pl.when(j == 0)
    def _init():
        buf_ref[...] = jnp.zeros_like(buf_ref)

    def poke(row_ref, c, value):
        base = pl.multiple_of((c // 128) * 128, 128)
        lane = jax.lax.broadcasted_iota(jnp.int32, (128,), 0)
        vec = jnp.where(lane == (c - base), value, 0.0).astype(jnp.float32)
        row_ref[pl.ds(base, 128)] = vec

    @pl.when(j >= _NSLOT)
    def _recycle():
        for s in range(_NSLOT):
            @pl.when(slot == s)
            def _():
                pltpu.make_async_copy(
                    buf_ref.at[s],
                    out_ref.at[pl.ds((j - _NSLOT) * _BB, _BB), :],
                    sems[s]).wait()
        for i in range(_BB):
            c = idx_ref[(j - _NSLOT) * _BB + i]
            poke(buf_ref.at[slot, i], c, 0.0)

    for i in range(_BB):
        c = idx_ref[j * _BB + i]
        poke(buf_ref.at[slot, i], c, 1.0)

    for s in range(_NSLOT):
        @pl.when(slot == s)
        def _():
            pltpu.make_async_copy(
                buf_ref.at[s],
                out_ref.at[pl.ds(j * _BB, _BB), :],
                sems[s]).start(priority=s % 2)

    @pl.when(j == nsteps - 1)
    def _drain():
        for t in range(_NSLOT):
            step = nsteps - _NSLOT + t
            pltpu.make_async_copy(
                buf_ref.at[step % _NSLOT],
                out_ref.at[pl.ds(step * _BB, _BB), :],
                sems[step % _NSLOT]).wait()


def kernel(indexes, weight):
    vocab = weight.shape[0]
    batch = indexes.shape[0]
    idx = indexes.astype(jnp.int32)
    grid_spec = pltpu.PrefetchScalarGridSpec(
        num_scalar_prefetch=1,
        grid=(batch // _BB,),
        in_specs=[],
        out_specs=pl.BlockSpec(memory_space=pl.ANY),
        scratch_shapes=[
            pltpu.VMEM((_NSLOT, _BB, vocab), jnp.float32),
        ] + [pltpu.SemaphoreType.DMA] * _NSLOT,
    )
    return pl.pallas_call(
        _onehot_dma,
        grid_spec=grid_spec,
        out_shape=jax.ShapeDtypeStruct((batch, vocab), jnp.float32),
    )(idx)
